# baseline (speedup 1.0000x reference)
import functools

import jax
import jax.numpy as jnp
from jax import lax
from jax.experimental import pallas as pl
from jax.experimental.pallas import tpu as pltpu
from jax.experimental.pallas import tpu_sc as plsc

N = 4096
ROLE = 1024
OUT_CH = 64
NROWS = 28
BN = 512
NI = N // BN


def _sc_counts_body(lead_hbm, nonm_hbm, memb_hbm, out_hbm, idx_v, acc_v):
    cid = lax.axis_index("c")
    sid = lax.axis_index("s")
    wid = sid * 2 + cid

    @pl.when(wid == 0)
    def _():
        pltpu.sync_copy(lead_hbm, idx_v)

    @pl.when(wid == 1)
    def _():
        pltpu.sync_copy(nonm_hbm, idx_v)

    @pl.when(wid == 2)
    def _():
        pltpu.sync_copy(memb_hbm, idx_v)

    @pl.when(wid < 3)
    def _():
        zeros16 = jnp.zeros((16,), jnp.float32)

        def zero_body(j, carry):
            for u in range(8):
                acc_v[pl.ds(j * 128 + u * 16, 16)] = zeros16
            return carry

        lax.fori_loop(0, N // 128, zero_body, 0)

        ones16 = jnp.ones((16,), jnp.float32)

        def scat_body(j, carry):
            iv = idx_v[pl.ds(j * 16, 16)]
            plsc.addupdate_scatter(acc_v, [iv], ones16)
            return carry

        lax.fori_loop(0, ROLE // 16, scat_body, 0)

        pltpu.sync_copy(acc_v, out_hbm.at[wid])


def _sc_counts(leaders, nonmembers, members):
    return pl.kernel(
        _sc_counts_body,
        out_type=jax.ShapeDtypeStruct((3, N), jnp.float32),
        mesh=plsc.VectorSubcoreMesh(core_axis_name="c", subcore_axis_name="s"),
        scratch_types=[
            pltpu.VMEM((ROLE,), jnp.int32),
            pltpu.VMEM((N,), jnp.float32),
        ],
        compiler_params=pltpu.CompilerParams(needs_layout_passes=False),
    )(leaders, nonmembers, members)


def _tc_body(adjc_ref, ct_ref, f_ref, w2t_ref, bt_ref, out_ref, gt_ref,
             ft_ref):
    i = pl.program_id(0)

    @pl.when(i == 0)
    def _():
        ct = ct_ref[...]
        ft = f_ref[...].T
        ft_ref[...] = ft
        gt = jnp.concatenate(
            [ct[0:1, :] * ft, ct[1:2, :] * ft, ct[2:3, :] * ft], axis=0
        ) * (1.0 / ROLE)
        gt_ref[...] = gt.astype(jnp.bfloat16)

    adjc = adjc_ref[...]

    acc = lax.dot_general(
        gt_ref[...], adjc.astype(jnp.bfloat16),
        (((1,), (0,)), ((), ())),
        preferred_element_type=jnp.float32,
    )

    dblk = adjc_ref[pl.ds(i * BN, BN), :]
    rows = lax.broadcasted_iota(jnp.int32, (BN, BN), 0)
    cols = lax.broadcasted_iota(jnp.int32, (BN, BN), 1)
    diag = jnp.sum(
        jnp.where(rows == cols, dblk, 0.0), axis=0, keepdims=True
    )

    git = gt_ref[:, pl.ds(i * BN, BN)].astype(jnp.float32)
    ht = acc - diag * git

    fit = ft_ref[:, pl.ds(i * BN, BN)]
    h0t = jnp.concatenate([ht, fit], axis=0)
    out = lax.dot_general(
        w2t_ref[...], h0t.astype(jnp.bfloat16),
        (((1,), (0,)), ((), ())),
        preferred_element_type=jnp.float32,
    )
    out_ref[...] = out + bt_ref[...]


def _tc_call(adj, counts, f2, w2t, bt):
    return pl.pallas_call(
        _tc_body,
        grid=(NI,),
        in_specs=[
            pl.BlockSpec((N, BN), lambda i: (0, i)),
            pl.BlockSpec((3, N), lambda i: (0, 0)),
            pl.BlockSpec((N, 7), lambda i: (0, 0)),
            pl.BlockSpec((NROWS * OUT_CH, NROWS), lambda i: (0, 0)),
            pl.BlockSpec((NROWS * OUT_CH, 1), lambda i: (0, 0)),
        ],
        out_specs=pl.BlockSpec((NROWS * OUT_CH, BN), lambda i: (0, i)),
        out_shape=jax.ShapeDtypeStruct((NROWS * OUT_CH, N), jnp.float32),
        scratch_shapes=[
            pltpu.VMEM((21, N), jnp.bfloat16),
            pltpu.VMEM((7, N), jnp.float32),
        ],
    )(adj, counts, f2, w2t, bt)


def kernel(feature, adj, members, nonmembers, leaders, weight, bias):
    f2 = feature.reshape(N, 7)
    counts = _sc_counts(
        leaders.astype(jnp.int32),
        nonmembers.astype(jnp.int32),
        members.astype(jnp.int32),
    )
    w2t = jnp.kron(jnp.eye(NROWS, dtype=weight.dtype), weight).T.astype(
        jnp.bfloat16)
    bt = jnp.tile(bias, NROWS).reshape(NROWS * OUT_CH, 1)
    out_t = _tc_call(adj, counts, f2, w2t, bt)
    return out_t.reshape(NROWS, OUT_CH, N).transpose(2, 0, 1)


# speedup vs baseline: 1.0425x; 1.0425x over previous
import functools

import jax
import jax.numpy as jnp
from jax import lax
from jax.experimental import pallas as pl
from jax.experimental.pallas import tpu as pltpu
from jax.experimental.pallas import tpu_sc as plsc

N = 4096
ROLE = 1024
OUT_CH = 64
NROWS = 28
BN = 512
NI = N // BN


def _sc_counts_body(lead_hbm, nonm_hbm, memb_hbm, out_hbm, idx_v, acc_v):
    cid = lax.axis_index("c")
    sid = lax.axis_index("s")
    wid = sid * 2 + cid

    @pl.when(wid == 0)
    def _():
        pltpu.sync_copy(lead_hbm, idx_v)

    @pl.when(wid == 1)
    def _():
        pltpu.sync_copy(nonm_hbm, idx_v)

    @pl.when(wid == 2)
    def _():
        pltpu.sync_copy(memb_hbm, idx_v)

    @pl.when(wid < 3)
    def _():
        zeros16 = jnp.zeros((16,), jnp.float32)

        def zero_body(j, carry):
            for u in range(8):
                acc_v[pl.ds(j * 128 + u * 16, 16)] = zeros16
            return carry

        lax.fori_loop(0, N // 128, zero_body, 0)

        ones16 = jnp.ones((16,), jnp.float32)

        def scat_body(j, carry):
            iv = idx_v[pl.ds(j * 16, 16)]
            plsc.addupdate_scatter(acc_v, [iv], ones16)
            return carry

        lax.fori_loop(0, ROLE // 16, scat_body, 0)

        pltpu.sync_copy(acc_v, out_hbm.at[wid])


def _sc_counts(leaders, nonmembers, members):
    return pl.kernel(
        _sc_counts_body,
        out_type=jax.ShapeDtypeStruct((3, N), jnp.float32),
        mesh=plsc.VectorSubcoreMesh(core_axis_name="c", subcore_axis_name="s"),
        scratch_types=[
            pltpu.VMEM((ROLE,), jnp.int32),
            pltpu.VMEM((N,), jnp.float32),
        ],
        compiler_params=pltpu.CompilerParams(needs_layout_passes=False),
    )(leaders, nonmembers, members)


def _tc_body(adjc_ref, ct_ref, ft_ref, w2t_ref, bt_ref, out_ref, gt_ref):
    i = pl.program_id(0)

    @pl.when(i == 0)
    def _():
        ct = ct_ref[...]
        ft = ft_ref[...]
        gt = jnp.concatenate(
            [ct[0:1, :] * ft, ct[1:2, :] * ft, ct[2:3, :] * ft], axis=0
        ) * (1.0 / ROLE)
        gt_ref[...] = gt.astype(jnp.bfloat16)

    adjc = adjc_ref[...]

    acc = lax.dot_general(
        gt_ref[...], adjc.astype(jnp.bfloat16),
        (((1,), (0,)), ((), ())),
        preferred_element_type=jnp.float32,
    )

    dblk = adjc_ref[pl.ds(i * BN, BN), :]
    rows = lax.broadcasted_iota(jnp.int32, (BN, BN), 0)
    cols = lax.broadcasted_iota(jnp.int32, (BN, BN), 1)
    diag = jnp.sum(
        jnp.where(rows == cols, dblk, 0.0), axis=0, keepdims=True
    )

    git = gt_ref[:, pl.ds(i * BN, BN)].astype(jnp.float32)
    ht = acc - diag * git

    fit = ft_ref[:, pl.ds(i * BN, BN)]
    h0t = jnp.concatenate([ht, fit], axis=0)
    out = lax.dot_general(
        w2t_ref[...], h0t.astype(jnp.bfloat16),
        (((1,), (0,)), ((), ())),
        preferred_element_type=jnp.float32,
    )
    out_ref[...] = out + bt_ref[...]


def _tc_call(adj, counts, ft, w2t, bt):
    return pl.pallas_call(
        _tc_body,
        grid=(NI,),
        in_specs=[
            pl.BlockSpec((N, BN), lambda i: (0, i)),
            pl.BlockSpec((3, N), lambda i: (0, 0)),
            pl.BlockSpec((7, N), lambda i: (0, 0)),
            pl.BlockSpec((NROWS * OUT_CH, NROWS), lambda i: (0, 0)),
            pl.BlockSpec((NROWS * OUT_CH, 1), lambda i: (0, 0)),
        ],
        out_specs=pl.BlockSpec((NROWS * OUT_CH, BN), lambda i: (0, i)),
        out_shape=jax.ShapeDtypeStruct((NROWS * OUT_CH, N), jnp.float32),
        scratch_shapes=[pltpu.VMEM((21, N), jnp.bfloat16)],
    )(adj, counts, ft, w2t, bt)


def kernel(feature, adj, members, nonmembers, leaders, weight, bias):
    ft = feature.reshape(N, 7).T
    counts = _sc_counts(
        leaders.astype(jnp.int32),
        nonmembers.astype(jnp.int32),
        members.astype(jnp.int32),
    )
    w2t = jnp.kron(jnp.eye(NROWS, dtype=weight.dtype), weight).T.astype(
        jnp.bfloat16)
    bt = jnp.tile(bias, NROWS).reshape(NROWS * OUT_CH, 1)
    out_t = _tc_call(adj, counts, ft, w2t, bt)
    return out_t.reshape(NROWS, OUT_CH, N).transpose(2, 0, 1)
